# Initial kernel scaffold; baseline (speedup 1.0000x reference)
import jax
import jax.numpy as jnp
from jax.experimental import pallas as pl


def kernel(input, table, gamma, beta):
    raise NotImplementedError("write your pallas kernel here")


# speedup vs baseline: 1.0227x; 1.0227x over previous
import functools

import jax
import jax.numpy as jnp
from jax import lax
from jax.experimental import pallas as pl
from jax.experimental.pallas import tpu as pltpu
from jax.experimental.pallas import tpu_sc as plsc

D = 64
L = 16
EPS = 1e-5


def _splat_sum(v):
    iota = lax.broadcasted_iota(jnp.int32, (L,), 0)
    for sh in (8, 4, 2, 1):
        v = v + v.at[iota ^ sh].get(mode="promise_in_bounds")
    return v


def _rsqrt(x):
    i = lax.bitcast_convert_type(x, jnp.int32)
    i = jnp.int32(0x5F3759DF) - (i >> 1)
    y = lax.bitcast_convert_type(i, jnp.float32)
    xh = x * 0.5
    y = y * (1.5 - xh * y * y)
    y = y * (1.5 - xh * y * y)
    y = y * (1.5 - xh * y * y)
    return y


@functools.partial(jax.jit, static_argnames=("n_chunks", "chunk"))
def _embed_ln(table, idx, gamma, beta, n_chunks, chunk):
    B = idx.shape[0]
    NC, NS = 2, 16
    NW = NC * NS
    b_per_w = B // NW

    mesh = plsc.VectorSubcoreMesh(core_axis_name="c", subcore_axis_name="s")

    @functools.partial(
        pl.kernel,
        mesh=mesh,
        out_type=jax.ShapeDtypeStruct((B, D), jnp.float32),
        scratch_types=[
            pltpu.VMEM((chunk,), jnp.int32),
            pltpu.VMEM((chunk, D), jnp.float32),
            pltpu.VMEM((D,), jnp.float32),
            pltpu.VMEM((D,), jnp.float32),
            pltpu.SemaphoreType.DMA,
        ],
        compiler_params=pltpu.CompilerParams(use_tc_tiling_on_sc=False),
    )
    def k(table_hbm, idx_hbm, gamma_hbm, beta_hbm, out_hbm,
          idx_v, data_v, g_v, bt_v, sem):
        wid = lax.axis_index("s") * NC + lax.axis_index("c")
        base = wid * b_per_w

        pltpu.sync_copy(gamma_hbm, g_v)
        pltpu.sync_copy(beta_hbm, bt_v)
        g = [g_v[pl.ds(j * L, L)] for j in range(D // L)]
        bt = [bt_v[pl.ds(j * L, L)] for j in range(D // L)]

        def chunk_body(ci, carry):
            off = base + ci * chunk
            pltpu.sync_copy(idx_hbm.at[pl.ds(off, chunk)], idx_v)
            pltpu.async_copy(table_hbm.at[idx_v], data_v, sem).wait()

            def row_body(r, c2):
                x = [data_v[r, pl.ds(j * L, L)] for j in range(D // L)]
                tot = _splat_sum((x[0] + x[1]) + (x[2] + x[3]))
                mean = tot * (1.0 / D)
                a = [xj - mean for xj in x]
                sq = (a[0] * a[0] + a[1] * a[1]) + (a[2] * a[2] + a[3] * a[3])
                var = _splat_sum(sq) * (1.0 / D)
                rstd = _rsqrt(var + EPS)
                for j in range(D // L):
                    data_v[r, pl.ds(j * L, L)] = a[j] * rstd * g[j] + bt[j]
                return c2

            lax.fori_loop(0, chunk, row_body, 0)
            pltpu.sync_copy(data_v, out_hbm.at[pl.ds(off, chunk)])
            return carry

        lax.fori_loop(0, n_chunks, chunk_body, 0)

    return k(table, idx, gamma, beta)


def kernel(input, table, gamma, beta):
    idx = input.reshape(-1).astype(jnp.int32)
    B = idx.shape[0]
    chunk = 512
    n_chunks = B // (32 * chunk)
    out = _embed_ln(table, idx, gamma, beta, n_chunks, chunk)
    return out.reshape(input.shape + (D,))


# speedup vs baseline: 1.8412x; 1.8003x over previous
import functools

import jax
import jax.numpy as jnp
from jax import lax
from jax.experimental import pallas as pl
from jax.experimental.pallas import tpu as pltpu
from jax.experimental.pallas import tpu_sc as plsc

D = 64
L = 16
EPS = 1e-5
NBUF = 2
UNROLL = 4


def _splat_sum(v):
    iota = lax.broadcasted_iota(jnp.int32, (L,), 0)
    for sh in (8, 4, 2, 1):
        v = v + v.at[iota ^ sh].get(mode="promise_in_bounds")
    return v


def _rsqrt(x):
    i = lax.bitcast_convert_type(x, jnp.int32)
    i = jnp.int32(0x5F3759DF) - (i >> 1)
    y = lax.bitcast_convert_type(i, jnp.float32)
    xh = x * 0.5
    y = y * (1.5 - xh * y * y)
    y = y * (1.5 - xh * y * y)
    return y


def _ln_row(data, r, g, bt):
    x = [data[r, pl.ds(j * L, L)] for j in range(D // L)]
    s = (x[0] + x[1]) + (x[2] + x[3])
    q = ((x[0] * x[0] + x[1] * x[1]) + (x[2] * x[2] + x[3] * x[3]))
    mean = _splat_sum(s) * (1.0 / D)
    ex2 = _splat_sum(q) * (1.0 / D) + EPS
    rstd = _rsqrt(ex2 - mean * mean)
    for j in range(D // L):
        data[r, pl.ds(j * L, L)] = (x[j] - mean) * rstd * g[j] + bt[j]


@functools.partial(jax.jit, static_argnames=("n_chunks", "chunk"))
def _embed_ln(table, idx, gamma, beta, n_chunks, chunk):
    B = idx.shape[0]
    NC, NS = 2, 16
    NW = NC * NS
    b_per_w = B // NW
    n_pairs = n_chunks // NBUF

    mesh = plsc.VectorSubcoreMesh(core_axis_name="c", subcore_axis_name="s")

    @functools.partial(
        pl.kernel,
        mesh=mesh,
        out_type=jax.ShapeDtypeStruct((B, D), jnp.float32),
        scratch_types=[
            pltpu.VMEM((b_per_w,), jnp.int32),
            pltpu.VMEM((NBUF, chunk, D), jnp.float32),
            pltpu.VMEM((D,), jnp.float32),
            pltpu.VMEM((D,), jnp.float32),
            pltpu.SemaphoreType.DMA((NBUF,)),
            pltpu.SemaphoreType.DMA((NBUF,)),
        ],
        compiler_params=pltpu.CompilerParams(use_tc_tiling_on_sc=False),
    )
    def k(table_hbm, idx_hbm, gamma_hbm, beta_hbm, out_hbm,
          idx_v, data_v, g_v, bt_v, gsem, osem):
        wid = lax.axis_index("s") * NC + lax.axis_index("c")
        base = wid * b_per_w

        pltpu.sync_copy(idx_hbm.at[pl.ds(base, b_per_w)], idx_v)
        pltpu.sync_copy(gamma_hbm, g_v)
        pltpu.sync_copy(beta_hbm, bt_v)
        g = [g_v[pl.ds(j * L, L)] for j in range(D // L)]
        bt = [bt_v[pl.ds(j * L, L)] for j in range(D // L)]

        def gather(ci, b):
            return pltpu.make_async_copy(
                table_hbm.at[idx_v.at[pl.ds(ci * chunk, chunk)]],
                data_v.at[b], gsem.at[b])

        def writeback(ci, b):
            return pltpu.make_async_copy(
                data_v.at[b], out_hbm.at[pl.ds(base + ci * chunk, chunk)],
                osem.at[b])

        for b in range(NBUF):
            gather(b, b).start()

        def pair_body(gi, carry):
            for b in range(NBUF):
                ci = gi * NBUF + b
                gather(ci, b).wait()

                def rows_body(t, c2):
                    r0 = t * UNROLL
                    for u in range(UNROLL):
                        _ln_row(data_v.at[b], r0 + u, g, bt)
                    return c2

                lax.fori_loop(0, chunk // UNROLL, rows_body, 0)
                writeback(ci, b).start()

                @pl.when(gi < n_pairs - 1)
                def _():
                    writeback(ci, b).wait()
                    gather(ci + NBUF, b).start()
            return carry

        lax.fori_loop(0, n_pairs, pair_body, 0)
        for b in range(NBUF):
            writeback(n_chunks - NBUF + b, b).wait()

    return k(table, idx, gamma, beta)


def kernel(input, table, gamma, beta):
    idx = input.reshape(-1).astype(jnp.int32)
    B = idx.shape[0]
    chunk = 512
    n_chunks = B // (32 * chunk)
    out = _embed_ln(table, idx, gamma, beta, n_chunks, chunk)
    return out.reshape(input.shape + (D,))
